# baseline (speedup 1.0000x reference)
import functools

import jax
import jax.numpy as jnp
from jax import lax
from jax.experimental import pallas as pl
from jax.experimental.pallas import tpu as pltpu
from jax.experimental.pallas import tpu_sc as plsc

N_NODES = 10000
N_EDGES = 320000
D_FEAT = 128

NUM_CORES = 2
NUM_SUBCORES = 16
NW = NUM_CORES * NUM_SUBCORES

CHUNK = 120
ROWS_PER_TILE = 632
N_PAD = NUM_SUBCORES * ROWS_PER_TILE

NBUF = 3
NGRP0 = 36
NGRP1 = 20
GRP_EDGES = NBUF * CHUNK
E_CORE0 = NUM_SUBCORES * NGRP0 * GRP_EDGES
E_CORE1 = NUM_SUBCORES * NGRP1 * GRP_EDGES
E_PAD = E_CORE0 + E_CORE1

ROW_BLOCK = 2000
N_BLOCKS = N_NODES // ROW_BLOCK


def _sc_agg_body(x_hbm, sd_hbm, zeros_hbm, agg_hbm,
                 sd_v, rows_v, agg_sh, *sems):
    gsem = sems[:NBUF]
    ssem = sems[NBUF:2 * NBUF]
    isem = sems[2 * NBUF]
    c = lax.axis_index("c")
    s = lax.axis_index("s")
    w = c * NUM_SUBCORES + s
    ngrp = jnp.where(c == 0, NGRP0, NGRP1)

    pltpu.sync_copy(sd_hbm.at[w, 0], sd_v.at[0])
    pltpu.async_copy(sd_hbm.at[w, 1], sd_v.at[1], isem)
    for b in range(NBUF):
        pltpu.async_copy(x_hbm.at[sd_v.at[0, 0, b]], rows_v.at[b], gsem[b])
    pltpu.sync_copy(zeros_hbm, agg_sh.at[pl.ds(s * ROWS_PER_TILE, ROWS_PER_TILE)])
    plsc.subcore_barrier()

    def grp(g, carry):
        p = g & 1
        q = 1 - p
        pltpu.make_async_copy(sd_hbm.at[w, g], sd_v.at[q], isem).wait()
        for b in range(NBUF):
            pltpu.make_async_copy(
                x_hbm.at[sd_v.at[p, 0, b]], rows_v.at[b], gsem[b]).wait()
            pltpu.async_copy(
                rows_v.at[b], agg_sh.at[sd_v.at[p, 1, b]], ssem[b], add=True)
            pltpu.make_async_copy(
                rows_v.at[b], agg_sh.at[sd_v.at[p, 1, b]], ssem[b]).wait()
            pltpu.async_copy(
                x_hbm.at[sd_v.at[q, 0, b]], rows_v.at[b], gsem[b])
        gnext = jnp.minimum(g + 2, ngrp - 1)
        pltpu.async_copy(sd_hbm.at[w, gnext], sd_v.at[p], isem)
        return carry

    lax.fori_loop(0, ngrp - 1, grp, 0)

    pl_ = 1
    pltpu.make_async_copy(sd_hbm.at[w, 0], sd_v.at[1 - pl_], isem).wait()
    for b in range(NBUF):
        pltpu.make_async_copy(
            x_hbm.at[sd_v.at[pl_, 0, b]], rows_v.at[b], gsem[b]).wait()
        pltpu.sync_copy(rows_v.at[b], agg_sh.at[sd_v.at[pl_, 1, b]], add=True)
    plsc.subcore_barrier()

    pltpu.sync_copy(
        agg_sh.at[pl.ds(s * ROWS_PER_TILE, ROWS_PER_TILE)],
        agg_hbm.at[pl.ds(c * N_PAD + s * ROWS_PER_TILE, ROWS_PER_TILE)],
    )


_sc_agg = functools.partial(
    pl.kernel,
    out_type=jax.ShapeDtypeStruct((NUM_CORES * N_PAD, D_FEAT), jnp.float32),
    mesh=plsc.VectorSubcoreMesh(core_axis_name="c", subcore_axis_name="s"),
    scratch_types=[
        pltpu.VMEM((2, 2, NBUF, CHUNK), jnp.int32),
        pltpu.VMEM((NBUF, CHUNK, D_FEAT), jnp.float32),
        pltpu.VMEM_SHARED((N_PAD, D_FEAT), jnp.float32),
    ] + [pltpu.SemaphoreType.DMA] * (2 * NBUF + 1),
)(_sc_agg_body)


def _tc_body(agg_ref, x_ref, w_ref, ws_ref, b_ref, o_ref):
    a = agg_ref[0] + agg_ref[1]
    acc = jnp.dot(a, w_ref[...], preferred_element_type=jnp.float32)
    acc = acc + jnp.dot(x_ref[...], ws_ref[...], preferred_element_type=jnp.float32)
    acc = acc + b_ref[...]
    o_ref[...] = jnp.maximum(acc, 0.0)


@jax.jit
def kernel(x, edge_index, W, W_self, b):
    src = edge_index[0]
    dst = edge_index[1]
    pad = E_PAD - N_EDGES
    src_p = jnp.concatenate([src, jnp.zeros((pad,), jnp.int32)])
    dst_p = jnp.concatenate([dst, jnp.full((pad,), N_NODES, jnp.int32)])
    s0 = src_p[:E_CORE0].reshape(NUM_SUBCORES, NGRP0, NBUF, CHUNK)
    d0 = dst_p[:E_CORE0].reshape(NUM_SUBCORES, NGRP0, NBUF, CHUNK)
    s1 = src_p[E_CORE0:].reshape(NUM_SUBCORES, NGRP1, NBUF, CHUNK)
    d1 = dst_p[E_CORE0:].reshape(NUM_SUBCORES, NGRP1, NBUF, CHUNK)
    sd0 = jnp.stack([s0, d0], axis=2)
    sd1 = jnp.stack([s1, d1], axis=2)
    sd1 = jnp.pad(sd1, ((0, 0), (0, NGRP0 - NGRP1), (0, 0), (0, 0), (0, 0)))
    sd = jnp.concatenate([sd0, sd1], axis=0)
    zeros = jnp.zeros((ROWS_PER_TILE, D_FEAT), jnp.float32)

    agg = _sc_agg(x, sd, zeros)
    agg = agg.reshape(NUM_CORES, N_PAD, D_FEAT)

    out = pl.pallas_call(
        _tc_body,
        grid=(N_BLOCKS,),
        in_specs=[
            pl.BlockSpec((NUM_CORES, ROW_BLOCK, D_FEAT), lambda i: (0, i, 0)),
            pl.BlockSpec((ROW_BLOCK, D_FEAT), lambda i: (i, 0)),
            pl.BlockSpec((D_FEAT, D_FEAT), lambda i: (0, 0)),
            pl.BlockSpec((D_FEAT, D_FEAT), lambda i: (0, 0)),
            pl.BlockSpec((1, D_FEAT), lambda i: (0, 0)),
        ],
        out_specs=pl.BlockSpec((ROW_BLOCK, D_FEAT), lambda i: (i, 0)),
        out_shape=jax.ShapeDtypeStruct((N_NODES, D_FEAT), jnp.float32),
    )(agg, x, W, W_self, b.reshape(1, D_FEAT))
    return out


# speedup vs baseline: 1.0299x; 1.0299x over previous
import functools

import jax
import jax.numpy as jnp
from jax import lax
from jax.experimental import pallas as pl
from jax.experimental.pallas import tpu as pltpu
from jax.experimental.pallas import tpu_sc as plsc

N_NODES = 10000
N_EDGES = 320000
D_FEAT = 128

NUM_CORES = 2
NUM_SUBCORES = 16
NW = NUM_CORES * NUM_SUBCORES

CHUNK = 120
ROWS_PER_TILE = 632
N_PAD = NUM_SUBCORES * ROWS_PER_TILE

NBUF = 3
NGRP = 28
CHUNKS_PER_W = NGRP * NBUF
EDGES_PER_W_PAD = CHUNKS_PER_W * CHUNK
E_PAD = EDGES_PER_W_PAD * NW

ROW_BLOCK = 2000
N_BLOCKS = N_NODES // ROW_BLOCK


def _sc_agg_body(x_hbm, src_hbm, dst_hbm, zeros_hbm, agg_hbm,
                 sd_v, rows_v, agg_sh, *sems):
    gsem = sems[:NBUF]
    ssem = sems[NBUF:2 * NBUF]
    isem = sems[2 * NBUF]
    c = lax.axis_index("c")
    s = lax.axis_index("s")
    w = c * NUM_SUBCORES + s

    pltpu.sync_copy(src_hbm.at[w, 0], sd_v.at[0, 0])
    pltpu.sync_copy(dst_hbm.at[w, 0], sd_v.at[0, 1])
    pltpu.async_copy(src_hbm.at[w, 1], sd_v.at[1, 0], isem)
    pltpu.async_copy(dst_hbm.at[w, 1], sd_v.at[1, 1], isem)
    for b in range(NBUF):
        pltpu.async_copy(x_hbm.at[sd_v.at[0, 0, b]], rows_v.at[b], gsem[b])
    pltpu.sync_copy(zeros_hbm, agg_sh.at[pl.ds(s * ROWS_PER_TILE, ROWS_PER_TILE)])
    plsc.subcore_barrier()

    def grp(g, carry):
        p = g & 1
        q = 1 - p
        pltpu.make_async_copy(src_hbm.at[w, g], sd_v.at[q, 0], isem).wait()
        pltpu.make_async_copy(dst_hbm.at[w, g], sd_v.at[q, 1], isem).wait()
        for b in range(NBUF):
            pltpu.make_async_copy(
                x_hbm.at[sd_v.at[p, 0, b]], rows_v.at[b], gsem[b]).wait()
            pltpu.async_copy(
                rows_v.at[b], agg_sh.at[sd_v.at[p, 1, b]], ssem[b], add=True)
            pltpu.make_async_copy(
                rows_v.at[b], agg_sh.at[sd_v.at[p, 1, b]], ssem[b]).wait()
            pltpu.async_copy(
                x_hbm.at[sd_v.at[q, 0, b]], rows_v.at[b], gsem[b])
        gnext = jnp.minimum(g + 2, NGRP - 1)
        pltpu.async_copy(src_hbm.at[w, gnext], sd_v.at[p, 0], isem)
        pltpu.async_copy(dst_hbm.at[w, gnext], sd_v.at[p, 1], isem)
        return carry

    lax.fori_loop(0, NGRP - 1, grp, 0)

    pl_ = 1
    pltpu.make_async_copy(src_hbm.at[w, 0], sd_v.at[1 - pl_, 0], isem).wait()
    pltpu.make_async_copy(dst_hbm.at[w, 0], sd_v.at[1 - pl_, 1], isem).wait()
    for b in range(NBUF):
        pltpu.make_async_copy(
            x_hbm.at[sd_v.at[pl_, 0, b]], rows_v.at[b], gsem[b]).wait()
        pltpu.sync_copy(rows_v.at[b], agg_sh.at[sd_v.at[pl_, 1, b]], add=True)
    plsc.subcore_barrier()

    pltpu.sync_copy(
        agg_sh.at[pl.ds(s * ROWS_PER_TILE, ROWS_PER_TILE)],
        agg_hbm.at[pl.ds(c * N_PAD + s * ROWS_PER_TILE, ROWS_PER_TILE)],
    )


_sc_agg = functools.partial(
    pl.kernel,
    out_type=jax.ShapeDtypeStruct((NUM_CORES * N_PAD, D_FEAT), jnp.float32),
    mesh=plsc.VectorSubcoreMesh(core_axis_name="c", subcore_axis_name="s"),
    scratch_types=[
        pltpu.VMEM((2, 2, NBUF, CHUNK), jnp.int32),
        pltpu.VMEM((NBUF, CHUNK, D_FEAT), jnp.float32),
        pltpu.VMEM_SHARED((N_PAD, D_FEAT), jnp.float32),
    ] + [pltpu.SemaphoreType.DMA] * (2 * NBUF + 1),
)(_sc_agg_body)


def _tc_self_body(x_ref, ws_ref, b_ref, o_ref):
    o_ref[...] = jnp.dot(
        x_ref[...], ws_ref[...], preferred_element_type=jnp.float32) + b_ref[...]


def _tc_body(agg_ref, self_ref, w_ref, o_ref):
    a = agg_ref[0] + agg_ref[1]
    acc = jnp.dot(a, w_ref[...], preferred_element_type=jnp.float32)
    o_ref[...] = jnp.maximum(acc + self_ref[...], 0.0)


@jax.jit
def kernel(x, edge_index, W, W_self, b):
    src = edge_index[0]
    dst = edge_index[1]
    pad = E_PAD - N_EDGES
    src_p = jnp.concatenate([src, jnp.zeros((pad,), jnp.int32)])
    dst_p = jnp.concatenate([dst, jnp.full((pad,), N_NODES, jnp.int32)])
    src_w = src_p.reshape(NW, NGRP, NBUF, CHUNK)
    dst_w = dst_p.reshape(NW, NGRP, NBUF, CHUNK)
    zeros = jnp.zeros((ROWS_PER_TILE, D_FEAT), jnp.float32)

    self_out = pl.pallas_call(
        _tc_self_body,
        grid=(N_BLOCKS,),
        in_specs=[
            pl.BlockSpec((ROW_BLOCK, D_FEAT), lambda i: (i, 0)),
            pl.BlockSpec((D_FEAT, D_FEAT), lambda i: (0, 0)),
            pl.BlockSpec((1, D_FEAT), lambda i: (0, 0)),
        ],
        out_specs=pl.BlockSpec((ROW_BLOCK, D_FEAT), lambda i: (i, 0)),
        out_shape=jax.ShapeDtypeStruct((N_NODES, D_FEAT), jnp.float32),
    )(x, W_self, b.reshape(1, D_FEAT))

    agg = _sc_agg(x, src_w, dst_w, zeros)
    agg = agg.reshape(NUM_CORES, N_PAD, D_FEAT)

    out = pl.pallas_call(
        _tc_body,
        grid=(N_BLOCKS,),
        in_specs=[
            pl.BlockSpec((NUM_CORES, ROW_BLOCK, D_FEAT), lambda i: (0, i, 0)),
            pl.BlockSpec((ROW_BLOCK, D_FEAT), lambda i: (i, 0)),
            pl.BlockSpec((D_FEAT, D_FEAT), lambda i: (0, 0)),
        ],
        out_specs=pl.BlockSpec((ROW_BLOCK, D_FEAT), lambda i: (i, 0)),
        out_shape=jax.ShapeDtypeStruct((N_NODES, D_FEAT), jnp.float32),
    )(agg, self_out, W)
    return out


# speedup vs baseline: 1.6894x; 1.6403x over previous
import functools

import jax
import jax.numpy as jnp
from jax import lax
from jax.experimental import pallas as pl
from jax.experimental.pallas import tpu as pltpu
from jax.experimental.pallas import tpu_sc as plsc

N_NODES = 10000
N_EDGES = 320000
D_FEAT = 128

NUM_CORES = 2
NUM_SUBCORES = 16
NW = NUM_CORES * NUM_SUBCORES

CHUNK = 125
ROWS_PER_TILE = 632
N_PAD = NUM_SUBCORES * ROWS_PER_TILE

NBUF = 2
NGRP = 40
CHUNKS_PER_W = NGRP * NBUF

ROW_BLOCK = 2000
N_BLOCKS = N_NODES // ROW_BLOCK


def _sc_agg_body(x_hbm, ei_hbm, zeros_hbm, agg_hbm,
                 sd_v, rows_v, agg_sh, *sems):
    gsem = sems[:NBUF]
    ssem = sems[NBUF:2 * NBUF]
    isem = sems[2 * NBUF]
    c = lax.axis_index("c")
    s = lax.axis_index("s")
    w = c * NUM_SUBCORES + s

    pltpu.sync_copy(ei_hbm.at[0, w, 0], sd_v.at[0, 0])
    pltpu.sync_copy(ei_hbm.at[1, w, 0], sd_v.at[0, 1])
    pltpu.async_copy(ei_hbm.at[0, w, 1], sd_v.at[1, 0], isem)
    pltpu.async_copy(ei_hbm.at[1, w, 1], sd_v.at[1, 1], isem)
    for b in range(NBUF):
        pltpu.async_copy(x_hbm.at[sd_v.at[0, 0, b]], rows_v.at[b], gsem[b])
    pltpu.sync_copy(zeros_hbm, agg_sh.at[pl.ds(s * ROWS_PER_TILE, ROWS_PER_TILE)])
    plsc.subcore_barrier()

    def grp(g, carry):
        p = g & 1
        q = 1 - p
        pltpu.make_async_copy(ei_hbm.at[0, w, g], sd_v.at[q, 0], isem).wait()
        pltpu.make_async_copy(ei_hbm.at[1, w, g], sd_v.at[q, 1], isem).wait()
        for b in range(NBUF):
            pltpu.make_async_copy(
                x_hbm.at[sd_v.at[p, 0, b]], rows_v.at[b], gsem[b]).wait()
            pltpu.async_copy(
                rows_v.at[b], agg_sh.at[sd_v.at[p, 1, b]], ssem[b], add=True)
            pltpu.make_async_copy(
                rows_v.at[b], agg_sh.at[sd_v.at[p, 1, b]], ssem[b]).wait()
            pltpu.async_copy(
                x_hbm.at[sd_v.at[q, 0, b]], rows_v.at[b], gsem[b])
        gnext = jnp.minimum(g + 2, NGRP - 1)
        pltpu.async_copy(ei_hbm.at[0, w, gnext], sd_v.at[p, 0], isem)
        pltpu.async_copy(ei_hbm.at[1, w, gnext], sd_v.at[p, 1], isem)
        return carry

    lax.fori_loop(0, NGRP - 1, grp, 0)

    pl_ = 1
    pltpu.make_async_copy(ei_hbm.at[0, w, 0], sd_v.at[1 - pl_, 0], isem).wait()
    pltpu.make_async_copy(ei_hbm.at[1, w, 0], sd_v.at[1 - pl_, 1], isem).wait()
    for b in range(NBUF):
        pltpu.make_async_copy(
            x_hbm.at[sd_v.at[pl_, 0, b]], rows_v.at[b], gsem[b]).wait()
        pltpu.sync_copy(rows_v.at[b], agg_sh.at[sd_v.at[pl_, 1, b]], add=True)
    plsc.subcore_barrier()

    pltpu.sync_copy(
        agg_sh.at[pl.ds(s * ROWS_PER_TILE, ROWS_PER_TILE)],
        agg_hbm.at[pl.ds(c * N_PAD + s * ROWS_PER_TILE, ROWS_PER_TILE)],
    )


_sc_agg = functools.partial(
    pl.kernel,
    out_type=jax.ShapeDtypeStruct((NUM_CORES * N_PAD, D_FEAT), jnp.float32),
    mesh=plsc.VectorSubcoreMesh(core_axis_name="c", subcore_axis_name="s"),
    scratch_types=[
        pltpu.VMEM((2, 2, NBUF, CHUNK), jnp.int32),
        pltpu.VMEM((NBUF, CHUNK, D_FEAT), jnp.float32),
        pltpu.VMEM_SHARED((N_PAD, D_FEAT), jnp.float32),
    ] + [pltpu.SemaphoreType.DMA] * (2 * NBUF + 1),
)(_sc_agg_body)


def _tc_self_body(x_ref, ws_ref, b_ref, o_ref):
    o_ref[...] = jnp.dot(
        x_ref[...], ws_ref[...], preferred_element_type=jnp.float32) + b_ref[...]


def _tc_body(agg_ref, self_ref, w_ref, o_ref):
    a = agg_ref[0] + agg_ref[1]
    acc = jnp.dot(a, w_ref[...], preferred_element_type=jnp.float32)
    o_ref[...] = jnp.maximum(acc + self_ref[...], 0.0)


@jax.jit
def kernel(x, edge_index, W, W_self, b):
    ei_w = edge_index.reshape(2, NW, NGRP, NBUF, CHUNK)
    zeros = jnp.zeros((ROWS_PER_TILE, D_FEAT), jnp.float32)

    self_out = pl.pallas_call(
        _tc_self_body,
        grid=(N_BLOCKS,),
        in_specs=[
            pl.BlockSpec((ROW_BLOCK, D_FEAT), lambda i: (i, 0)),
            pl.BlockSpec((D_FEAT, D_FEAT), lambda i: (0, 0)),
            pl.BlockSpec((1, D_FEAT), lambda i: (0, 0)),
        ],
        out_specs=pl.BlockSpec((ROW_BLOCK, D_FEAT), lambda i: (i, 0)),
        out_shape=jax.ShapeDtypeStruct((N_NODES, D_FEAT), jnp.float32),
    )(x, W_self, b.reshape(1, D_FEAT))

    agg = _sc_agg(x, ei_w, zeros)
    agg = agg.reshape(NUM_CORES, N_PAD, D_FEAT)

    out = pl.pallas_call(
        _tc_body,
        grid=(N_BLOCKS,),
        in_specs=[
            pl.BlockSpec((NUM_CORES, ROW_BLOCK, D_FEAT), lambda i: (0, i, 0)),
            pl.BlockSpec((ROW_BLOCK, D_FEAT), lambda i: (i, 0)),
            pl.BlockSpec((D_FEAT, D_FEAT), lambda i: (0, 0)),
        ],
        out_specs=pl.BlockSpec((ROW_BLOCK, D_FEAT), lambda i: (i, 0)),
        out_shape=jax.ShapeDtypeStruct((N_NODES, D_FEAT), jnp.float32),
    )(agg, self_out, W)
    return out


# speedup vs baseline: 1.7784x; 1.0527x over previous
import functools

import jax
import jax.numpy as jnp
from jax import lax
from jax.experimental import pallas as pl
from jax.experimental.pallas import tpu as pltpu
from jax.experimental.pallas import tpu_sc as plsc

N_NODES = 10000
N_EDGES = 320000
D_FEAT = 128

NUM_CORES = 2
NUM_SUBCORES = 16
NW = NUM_CORES * NUM_SUBCORES

CHUNK = 128
NGRP = 39
NBUF = 2
E_W = NGRP * NBUF * CHUNK
EXTRA_BASE = NW * E_W

ROWS_PER_TILE = 632
N_PAD = NUM_SUBCORES * ROWS_PER_TILE

ROW_BLOCK = 2000
N_BLOCKS = N_NODES // ROW_BLOCK


def _sc_agg_body(x_hbm, ei_hbm, zeros_hbm, agg_hbm,
                 sd_v, sdt_v, rows_v, agg_sh, *sems):
    gsem = sems[:NBUF]
    ssem = sems[NBUF:2 * NBUF]
    isem = sems[2 * NBUF]
    c = lax.axis_index("c")
    s = lax.axis_index("s")
    w = c * NUM_SUBCORES + s
    wo = w * E_W

    def idx_copy(g, slot, fn):
        for b in range(NBUF):
            off = wo + g * (NBUF * CHUNK) + b * CHUNK
            fn(ei_hbm.at[pl.ds(0, 2), pl.ds(off, CHUNK)], sd_v.at[slot, b])

    idx_copy(0, 0, pltpu.sync_copy)
    idx_copy(1, 1, lambda a, v: pltpu.async_copy(a, v, isem))
    for b in range(NBUF):
        pltpu.async_copy(x_hbm.at[sd_v.at[0, b, 0]], rows_v.at[b], gsem[b])
    pltpu.sync_copy(zeros_hbm, agg_sh.at[pl.ds(s * ROWS_PER_TILE, ROWS_PER_TILE)])
    plsc.subcore_barrier()

    def grp(g, carry):
        p = g & 1
        q = 1 - p
        idx_copy(g, q, lambda a, v: pltpu.make_async_copy(a, v, isem).wait())
        for b in range(NBUF):
            pltpu.make_async_copy(
                x_hbm.at[sd_v.at[p, b, 0]], rows_v.at[b], gsem[b]).wait()
            pltpu.async_copy(
                rows_v.at[b], agg_sh.at[sd_v.at[p, b, 1]], ssem[b], add=True)
            pltpu.make_async_copy(
                rows_v.at[b], agg_sh.at[sd_v.at[p, b, 1]], ssem[b]).wait()
            pltpu.async_copy(
                x_hbm.at[sd_v.at[q, b, 0]], rows_v.at[b], gsem[b])
        gnext = jnp.minimum(g + 2, NGRP - 1)
        idx_copy(gnext, p, lambda a, v: pltpu.async_copy(a, v, isem))
        return carry

    lax.fori_loop(0, NGRP - 1, grp, 0)

    pl_ = (NGRP - 1) & 1
    idx_copy(0, 1 - pl_, lambda a, v: pltpu.make_async_copy(a, v, isem).wait())
    for b in range(NBUF):
        pltpu.make_async_copy(
            x_hbm.at[sd_v.at[pl_, b, 0]], rows_v.at[b], gsem[b]).wait()
        pltpu.sync_copy(rows_v.at[b], agg_sh.at[sd_v.at[pl_, b, 1]], add=True)

    @pl.when(w < (N_EDGES - EXTRA_BASE) // CHUNK)
    def _extra():
        eo = EXTRA_BASE + w * CHUNK
        pltpu.sync_copy(ei_hbm.at[pl.ds(0, 2), pl.ds(eo, CHUNK)], sdt_v)
        pltpu.sync_copy(x_hbm.at[sdt_v.at[0]], rows_v.at[0])
        pltpu.sync_copy(rows_v.at[0], agg_sh.at[sdt_v.at[1]], add=True)

    plsc.subcore_barrier()

    pltpu.sync_copy(
        agg_sh.at[pl.ds(s * ROWS_PER_TILE, ROWS_PER_TILE)],
        agg_hbm.at[pl.ds(c * N_PAD + s * ROWS_PER_TILE, ROWS_PER_TILE)],
    )


_sc_agg = functools.partial(
    pl.kernel,
    out_type=jax.ShapeDtypeStruct((NUM_CORES * N_PAD, D_FEAT), jnp.float32),
    mesh=plsc.VectorSubcoreMesh(core_axis_name="c", subcore_axis_name="s"),
    scratch_types=[
        pltpu.VMEM((2, NBUF, 2, CHUNK), jnp.int32),
        pltpu.VMEM((2, CHUNK), jnp.int32),
        pltpu.VMEM((NBUF, CHUNK, D_FEAT), jnp.float32),
        pltpu.VMEM_SHARED((N_PAD, D_FEAT), jnp.float32),
    ] + [pltpu.SemaphoreType.DMA] * (2 * NBUF + 1),
)(_sc_agg_body)


def _tc_self_body(x_ref, ws_ref, b_ref, o_ref):
    o_ref[...] = jnp.dot(
        x_ref[...], ws_ref[...], preferred_element_type=jnp.float32) + b_ref[...]


def _tc_body(agg_ref, self_ref, w_ref, o_ref):
    a = agg_ref[0] + agg_ref[1]
    acc = jnp.dot(a, w_ref[...], preferred_element_type=jnp.float32)
    o_ref[...] = jnp.maximum(acc + self_ref[...], 0.0)


@jax.jit
def kernel(x, edge_index, W, W_self, b):
    zeros = jnp.zeros((ROWS_PER_TILE, D_FEAT), jnp.float32)

    self_out = pl.pallas_call(
        _tc_self_body,
        grid=(N_BLOCKS,),
        in_specs=[
            pl.BlockSpec((ROW_BLOCK, D_FEAT), lambda i: (i, 0)),
            pl.BlockSpec((D_FEAT, D_FEAT), lambda i: (0, 0)),
            pl.BlockSpec((1, D_FEAT), lambda i: (0, 0)),
        ],
        out_specs=pl.BlockSpec((ROW_BLOCK, D_FEAT), lambda i: (i, 0)),
        out_shape=jax.ShapeDtypeStruct((N_NODES, D_FEAT), jnp.float32),
    )(x, W_self, b.reshape(1, D_FEAT))

    agg = _sc_agg(x, edge_index, zeros)
    agg = agg.reshape(NUM_CORES, N_PAD, D_FEAT)

    out = pl.pallas_call(
        _tc_body,
        grid=(N_BLOCKS,),
        in_specs=[
            pl.BlockSpec((NUM_CORES, ROW_BLOCK, D_FEAT), lambda i: (0, i, 0)),
            pl.BlockSpec((ROW_BLOCK, D_FEAT), lambda i: (i, 0)),
            pl.BlockSpec((D_FEAT, D_FEAT), lambda i: (0, 0)),
        ],
        out_specs=pl.BlockSpec((ROW_BLOCK, D_FEAT), lambda i: (i, 0)),
        out_shape=jax.ShapeDtypeStruct((N_NODES, D_FEAT), jnp.float32),
    )(agg, self_out, W)
    return out


# speedup vs baseline: 1.7818x; 1.0019x over previous
import functools

import jax
import jax.numpy as jnp
from jax import lax
from jax.experimental import pallas as pl
from jax.experimental.pallas import tpu as pltpu
from jax.experimental.pallas import tpu_sc as plsc

N_NODES = 10000
N_EDGES = 320000
D_FEAT = 128

NUM_CORES = 2
NUM_SUBCORES = 16
NW = NUM_CORES * NUM_SUBCORES

CHUNK = 128
NGRP = 39
NBUF = 2
E_W = NGRP * NBUF * CHUNK
EXTRA_BASE = NW * E_W

ROWS_PER_TILE = 632
N_PAD = NUM_SUBCORES * ROWS_PER_TILE

ROW_BLOCK = 2000
N_BLOCKS = N_NODES // ROW_BLOCK


def _sc_agg_body(x_hbm, ei_hbm, zeros_hbm, agg_hbm,
                 sd_v, sdt_v, rows_v, agg_sh, *sems):
    gsem = sems[:NBUF]
    ssem = sems[NBUF:2 * NBUF]
    isem = sems[2 * NBUF]
    c = lax.axis_index("c")
    s = lax.axis_index("s")
    w = c * NUM_SUBCORES + s
    wo = w * E_W

    def idx_copy(g, slot, fn):
        for b in range(NBUF):
            off = wo + g * (NBUF * CHUNK) + b * CHUNK
            fn(ei_hbm.at[pl.ds(0, 2), pl.ds(off, CHUNK)], sd_v.at[slot, b])

    idx_copy(0, 0, pltpu.sync_copy)
    idx_copy(1, 1, lambda a, v: pltpu.async_copy(a, v, isem))
    for b in range(NBUF):
        pltpu.async_copy(x_hbm.at[sd_v.at[0, b, 0]], rows_v.at[b], gsem[b])
    pltpu.sync_copy(zeros_hbm, agg_sh.at[pl.ds(s * ROWS_PER_TILE, ROWS_PER_TILE)])
    plsc.subcore_barrier()

    def grp(g, carry):
        p = g & 1
        q = 1 - p
        idx_copy(g, q, lambda a, v: pltpu.make_async_copy(a, v, isem).wait())
        for b in range(NBUF):
            pltpu.make_async_copy(
                x_hbm.at[sd_v.at[p, b, 0]], rows_v.at[b], gsem[b]).wait()
            pltpu.async_copy(
                rows_v.at[b], agg_sh.at[sd_v.at[p, b, 1]], ssem[b], add=True)
            pltpu.make_async_copy(
                rows_v.at[b], agg_sh.at[sd_v.at[p, b, 1]], ssem[b]).wait()
            pltpu.async_copy(
                x_hbm.at[sd_v.at[q, b, 0]], rows_v.at[b], gsem[b])
        gnext = jnp.minimum(g + 2, NGRP - 1)
        idx_copy(gnext, p, lambda a, v: pltpu.async_copy(a, v, isem))
        return carry

    lax.fori_loop(0, NGRP - 1, grp, 0)

    pl_ = (NGRP - 1) & 1
    idx_copy(0, 1 - pl_, lambda a, v: pltpu.make_async_copy(a, v, isem).wait())
    for b in range(NBUF):
        pltpu.make_async_copy(
            x_hbm.at[sd_v.at[pl_, b, 0]], rows_v.at[b], gsem[b]).wait()
        pltpu.sync_copy(rows_v.at[b], agg_sh.at[sd_v.at[pl_, b, 1]], add=True)

    @pl.when(s < (N_EDGES - EXTRA_BASE) // CHUNK // NUM_CORES)
    def _extra():
        eo = EXTRA_BASE + (c * 2 + s) * CHUNK
        pltpu.sync_copy(ei_hbm.at[pl.ds(0, 2), pl.ds(eo, CHUNK)], sdt_v)
        pltpu.sync_copy(x_hbm.at[sdt_v.at[0]], rows_v.at[0])
        pltpu.sync_copy(rows_v.at[0], agg_sh.at[sdt_v.at[1]], add=True)

    plsc.subcore_barrier()

    pltpu.sync_copy(
        agg_sh.at[pl.ds(s * ROWS_PER_TILE, ROWS_PER_TILE)],
        agg_hbm.at[pl.ds(c * N_PAD + s * ROWS_PER_TILE, ROWS_PER_TILE)],
    )


_sc_agg = functools.partial(
    pl.kernel,
    out_type=jax.ShapeDtypeStruct((NUM_CORES * N_PAD, D_FEAT), jnp.float32),
    mesh=plsc.VectorSubcoreMesh(core_axis_name="c", subcore_axis_name="s"),
    scratch_types=[
        pltpu.VMEM((2, NBUF, 2, CHUNK), jnp.int32),
        pltpu.VMEM((2, CHUNK), jnp.int32),
        pltpu.VMEM((NBUF, CHUNK, D_FEAT), jnp.float32),
        pltpu.VMEM_SHARED((N_PAD, D_FEAT), jnp.float32),
    ] + [pltpu.SemaphoreType.DMA] * (2 * NBUF + 1),
)(_sc_agg_body)


def _tc_self_body(x_ref, ws_ref, b_ref, o_ref):
    o_ref[...] = jnp.dot(
        x_ref[...], ws_ref[...], preferred_element_type=jnp.float32) + b_ref[...]


def _tc_body(agg_ref, self_ref, w_ref, o_ref):
    a = agg_ref[0] + agg_ref[1]
    acc = jnp.dot(a, w_ref[...], preferred_element_type=jnp.float32)
    o_ref[...] = jnp.maximum(acc + self_ref[...], 0.0)


@jax.jit
def kernel(x, edge_index, W, W_self, b):
    zeros = jnp.zeros((ROWS_PER_TILE, D_FEAT), jnp.float32)

    self_out = pl.pallas_call(
        _tc_self_body,
        grid=(N_BLOCKS,),
        in_specs=[
            pl.BlockSpec((ROW_BLOCK, D_FEAT), lambda i: (i, 0)),
            pl.BlockSpec((D_FEAT, D_FEAT), lambda i: (0, 0)),
            pl.BlockSpec((1, D_FEAT), lambda i: (0, 0)),
        ],
        out_specs=pl.BlockSpec((ROW_BLOCK, D_FEAT), lambda i: (i, 0)),
        out_shape=jax.ShapeDtypeStruct((N_NODES, D_FEAT), jnp.float32),
    )(x, W_self, b.reshape(1, D_FEAT))

    agg = _sc_agg(x, edge_index, zeros)
    agg = agg.reshape(NUM_CORES, N_PAD, D_FEAT)

    out = pl.pallas_call(
        _tc_body,
        grid=(N_BLOCKS,),
        in_specs=[
            pl.BlockSpec((NUM_CORES, ROW_BLOCK, D_FEAT), lambda i: (0, i, 0)),
            pl.BlockSpec((ROW_BLOCK, D_FEAT), lambda i: (i, 0)),
            pl.BlockSpec((D_FEAT, D_FEAT), lambda i: (0, 0)),
        ],
        out_specs=pl.BlockSpec((ROW_BLOCK, D_FEAT), lambda i: (i, 0)),
        out_shape=jax.ShapeDtypeStruct((N_NODES, D_FEAT), jnp.float32),
    )(agg, self_out, W)
    return out
